# baseline (speedup 1.0000x reference)
import functools

import jax
import jax.numpy as jnp
from jax import lax
from jax.experimental import pallas as pl
from jax.experimental.pallas import tpu as pltpu
from jax.experimental.pallas import tpu_sc as plsc

N_NODES = 10000
N_EDGES = 320000
D = 128

CHUNK = 128
NCHUNK = N_EDGES // CHUNK
NC = 2
NS = 16
NW = NC * NS
ROWS_PER_W = 80
NROWS_PAD = NW * ROWS_PER_W
NSTEP = ROWS_PER_W
DEPTH = 3
ROWS_PER_TILE = 624
TAIL_ROW0 = NS * ROWS_PER_TILE
TAIL_ROWS = N_NODES - TAIL_ROW0


def _sc_segment_sum_body(efeat_hbm, dst2d_hbm, zeros_hbm, out_hbm,
                         acc, bufs, idxs, sems, isems, ssems):
    c = lax.axis_index("c")
    s = lax.axis_index("s")
    wid = c * NS + s

    row0 = s * ROWS_PER_TILE
    pltpu.sync_copy(zeros_hbm.at[pl.ds(row0, ROWS_PER_TILE)],
                    acc.at[pl.ds(row0, ROWS_PER_TILE)])

    @pl.when(s == 0)
    def _():
        pltpu.sync_copy(zeros_hbm.at[pl.ds(TAIL_ROW0, TAIL_ROWS)],
                        acc.at[pl.ds(TAIL_ROW0, TAIL_ROWS)])

    plsc.subcore_barrier()

    lo_row = wid * ROWS_PER_W

    def gather(g):
        k = g % DEPTH
        row = lo_row + g
        pltpu.async_copy(dst2d_hbm.at[pl.ds(row, 1)], idxs[k], isems[k])
        pltpu.async_copy(efeat_hbm.at[pl.ds(row * CHUNK, CHUNK)],
                         bufs[k], sems[k])

    def wait(g):
        k = g % DEPTH
        row = lo_row + g
        pltpu.make_async_copy(dst2d_hbm.at[pl.ds(row, 1)], idxs[k],
                              isems[k]).wait()
        pltpu.make_async_copy(efeat_hbm.at[pl.ds(row * CHUNK, CHUNK)],
                              bufs[k], sems[k]).wait()

    def scatter_start(g):
        k = g % DEPTH
        pltpu.async_copy(bufs[k], acc.at[idxs[k].at[0]], ssems[k], add=True)

    def scatter_wait(g):
        k = g % DEPTH
        pltpu.make_async_copy(bufs[k], acc.at[idxs[k].at[0]],
                              ssems[k]).wait()

    def active(g):
        return lo_row + g < NCHUNK

    gather(0)
    for g in range(NSTEP):
        if g >= 2:
            @pl.when(active(g - 2))
            def _(g=g):
                scatter_wait(g - 2)

        if g + 1 < NSTEP:
            @pl.when(active(g + 1))
            def _(g=g):
                gather(g + 1)

        @pl.when(active(g))
        def _(g=g):
            wait(g)
            scatter_start(g)

    for t in range(max(0, NSTEP - 2), NSTEP):
        @pl.when(active(t))
        def _(t=t):
            scatter_wait(t)

    plsc.subcore_barrier()

    pltpu.sync_copy(acc.at[pl.ds(row0, ROWS_PER_TILE)],
                    out_hbm.at[pl.ds(c * N_NODES + row0, ROWS_PER_TILE)])

    @pl.when(s == 0)
    def _():
        pltpu.sync_copy(acc.at[pl.ds(TAIL_ROW0, TAIL_ROWS)],
                        out_hbm.at[pl.ds(c * N_NODES + TAIL_ROW0, TAIL_ROWS)])


@functools.cache
def _sc_segment_sum():
    return pl.kernel(
        _sc_segment_sum_body,
        mesh=plsc.VectorSubcoreMesh(core_axis_name="c", subcore_axis_name="s",
                                    num_cores=NC, num_subcores=NS),
        out_type=jax.ShapeDtypeStruct((NC * N_NODES, D), jnp.float32),
        scratch_types=[
            pltpu.VMEM_SHARED((N_NODES, D), jnp.float32),
            tuple(pltpu.VMEM((CHUNK, D), jnp.float32) for _ in range(DEPTH)),
            tuple(pltpu.VMEM((1, CHUNK), jnp.int32) for _ in range(DEPTH)),
            tuple(pltpu.SemaphoreType.DMA for _ in range(DEPTH)),
            tuple(pltpu.SemaphoreType.DMA for _ in range(DEPTH)),
            tuple(pltpu.SemaphoreType.DMA for _ in range(DEPTH)),
        ],
    )


def _copy_body(src, dst):
    dst[...] = src[...]


def _tc_copy(x):
    blk = 8000
    return pl.pallas_call(
        _copy_body,
        grid=(N_EDGES // blk,),
        in_specs=[pl.BlockSpec((blk, D), lambda i: (i, 0))],
        out_specs=pl.BlockSpec((blk, D), lambda i: (i, 0)),
        out_shape=jax.ShapeDtypeStruct((N_EDGES, D), jnp.float32),
    )(x)


def _mlp_body(p0, p1, nf, w1, b1, w2, b2, lns, lnb, out):
    agg = p0[...] + p1[...]
    x = (jnp.dot(agg, w1[0:D, :], preferred_element_type=jnp.float32)
         + jnp.dot(nf[...], w1[D:2 * D, :], preferred_element_type=jnp.float32)
         + b1[...])
    h = x * jax.nn.sigmoid(x)
    y = jnp.dot(h, w2[...], preferred_element_type=jnp.float32) + b2[...]
    mean = jnp.mean(y, axis=-1, keepdims=True)
    yc = y - mean
    var = jnp.mean(yc * yc, axis=-1, keepdims=True)
    out[...] = yc * lax.rsqrt(var + 1e-5) * lns[...] + lnb[...] + nf[...]


def _tc_mlp(part, nfeat, W1, b1, W2, b2, ln_scale, ln_bias):
    blk = 2000
    grid = (N_NODES // blk,)
    full = lambda shape: pl.BlockSpec(shape, lambda i: (0, 0))
    return pl.pallas_call(
        _mlp_body,
        grid=grid,
        in_specs=[
            pl.BlockSpec((blk, D), lambda i: (i, 0)),
            pl.BlockSpec((blk, D), lambda i: (i + N_NODES // blk, 0)),
            pl.BlockSpec((blk, D), lambda i: (i, 0)),
            full((2 * D, D)),
            full((1, D)),
            full((D, D)),
            full((1, D)),
            full((1, D)),
            full((1, D)),
        ],
        out_specs=pl.BlockSpec((blk, D), lambda i: (i, 0)),
        out_shape=jax.ShapeDtypeStruct((N_NODES, D), jnp.float32),
    )(part, part, nfeat, W1, b1, W2, b2, ln_scale, ln_bias)


def kernel(efeat, nfeat, edge_index, W1, b1, W2, b2, ln_scale, ln_bias):
    dst = edge_index[1]
    dst2d = jnp.concatenate(
        [dst, jnp.zeros((NROWS_PAD * CHUNK - N_EDGES,), jnp.int32)]
    ).reshape(NROWS_PAD, CHUNK)
    zeros = jnp.zeros((N_NODES, D), jnp.float32)
    part = _sc_segment_sum()(efeat, dst2d, zeros)
    nfeat_new = _tc_mlp(part, nfeat, W1, b1.reshape(1, D), W2,
                        b2.reshape(1, D), ln_scale.reshape(1, D),
                        ln_bias.reshape(1, D))
    return (_tc_copy(efeat), nfeat_new)


# speedup vs baseline: 1.0962x; 1.0962x over previous
import functools

import jax
import jax.numpy as jnp
from jax import lax
from jax.experimental import pallas as pl
from jax.experimental.pallas import tpu as pltpu
from jax.experimental.pallas import tpu_sc as plsc

N_NODES = 10000
N_EDGES = 320000
D = 128

CHUNK = 128
NCHUNK = N_EDGES // CHUNK
NC = 2
NS = 16
NW = NC * NS
ROWS_PER_W = 80
NROWS_PAD = NW * ROWS_PER_W
NSTEP = ROWS_PER_W
DEPTH = 3
ROWS_PER_TILE = 624
TAIL_ROW0 = NS * ROWS_PER_TILE
TAIL_ROWS = N_NODES - TAIL_ROW0


def _sc_segment_sum_body(efeat_hbm, dst3d_hbm, out_hbm,
                         acc, bufs, idxs, sems, isems, ssems):
    c = lax.axis_index("c")
    s = lax.axis_index("s")
    wid = c * NS + s

    lo_row = wid * ROWS_PER_W

    def gather(g):
        k = g % DEPTH
        row = lo_row + g
        pltpu.async_copy(dst3d_hbm.at[1, pl.ds(row, 1)], idxs[k], isems[k])
        pltpu.async_copy(efeat_hbm.at[pl.ds(row * CHUNK, CHUNK)],
                         bufs[k], sems[k])

    def wait(g):
        k = g % DEPTH
        row = lo_row + g
        pltpu.make_async_copy(dst3d_hbm.at[1, pl.ds(row, 1)], idxs[k],
                              isems[k]).wait()
        pltpu.make_async_copy(efeat_hbm.at[pl.ds(row * CHUNK, CHUNK)],
                              bufs[k], sems[k]).wait()

    gather(0)

    zbuf = bufs[DEPTH - 1]

    def zfill(r, carry):
        for j in range(D // 16):
            zbuf[r, pl.ds(16 * j, 16)] = jnp.zeros((16,), jnp.float32)
        return carry

    lax.fori_loop(0, CHUNK, zfill, 0)
    row0 = s * ROWS_PER_TILE
    for b in range(ROWS_PER_TILE // CHUNK):
        pltpu.sync_copy(zbuf, acc.at[pl.ds(row0 + b * CHUNK, CHUNK)])
    rem = ROWS_PER_TILE % CHUNK
    pltpu.sync_copy(zbuf.at[pl.ds(0, rem)],
                    acc.at[pl.ds(row0 + ROWS_PER_TILE - rem, rem)])

    @pl.when(s == 0)
    def _():
        pltpu.sync_copy(zbuf.at[pl.ds(0, TAIL_ROWS)],
                        acc.at[pl.ds(TAIL_ROW0, TAIL_ROWS)])

    plsc.subcore_barrier()

    def scatter_start(g):
        k = g % DEPTH
        pltpu.async_copy(bufs[k], acc.at[idxs[k].at[0]], ssems[k], add=True)

    def scatter_wait(g):
        k = g % DEPTH
        pltpu.make_async_copy(bufs[k], acc.at[idxs[k].at[0]],
                              ssems[k]).wait()

    def active(g):
        return lo_row + g < NCHUNK

    for g in range(NSTEP):
        if g >= 2:
            @pl.when(active(g - 2))
            def _(g=g):
                scatter_wait(g - 2)

        if g + 1 < NSTEP:
            @pl.when(active(g + 1))
            def _(g=g):
                gather(g + 1)

        @pl.when(active(g))
        def _(g=g):
            wait(g)
            scatter_start(g)

    for t in range(max(0, NSTEP - 2), NSTEP):
        @pl.when(active(t))
        def _(t=t):
            scatter_wait(t)

    plsc.subcore_barrier()

    pltpu.sync_copy(acc.at[pl.ds(row0, ROWS_PER_TILE)],
                    out_hbm.at[pl.ds(c * N_NODES + row0, ROWS_PER_TILE)])

    @pl.when(s == 0)
    def _():
        pltpu.sync_copy(acc.at[pl.ds(TAIL_ROW0, TAIL_ROWS)],
                        out_hbm.at[pl.ds(c * N_NODES + TAIL_ROW0, TAIL_ROWS)])


@functools.cache
def _sc_segment_sum():
    return pl.kernel(
        _sc_segment_sum_body,
        mesh=plsc.VectorSubcoreMesh(core_axis_name="c", subcore_axis_name="s",
                                    num_cores=NC, num_subcores=NS),
        out_type=jax.ShapeDtypeStruct((NC * N_NODES, D), jnp.float32),
        scratch_types=[
            pltpu.VMEM_SHARED((N_NODES, D), jnp.float32),
            tuple(pltpu.VMEM((CHUNK, D), jnp.float32) for _ in range(DEPTH)),
            tuple(pltpu.VMEM((1, CHUNK), jnp.int32) for _ in range(DEPTH)),
            tuple(pltpu.SemaphoreType.DMA for _ in range(DEPTH)),
            tuple(pltpu.SemaphoreType.DMA for _ in range(DEPTH)),
            tuple(pltpu.SemaphoreType.DMA for _ in range(DEPTH)),
        ],
    )


def _copy_body(src, dst):
    dst[...] = src[...]


def _tc_copy(x):
    blk = 8000
    return pl.pallas_call(
        _copy_body,
        grid=(N_EDGES // blk,),
        in_specs=[pl.BlockSpec((blk, D), lambda i: (i, 0))],
        out_specs=pl.BlockSpec((blk, D), lambda i: (i, 0)),
        out_shape=jax.ShapeDtypeStruct((N_EDGES, D), jnp.float32),
    )(x)


def _mlp_body(p0, p1, nf, w1, b1, w2, b2, lns, lnb, out):
    agg = p0[...] + p1[...]
    x = (jnp.dot(agg, w1[0:D, :], preferred_element_type=jnp.float32)
         + jnp.dot(nf[...], w1[D:2 * D, :], preferred_element_type=jnp.float32)
         + b1[...])
    h = x * jax.nn.sigmoid(x)
    y = jnp.dot(h, w2[...], preferred_element_type=jnp.float32) + b2[...]
    mean = jnp.mean(y, axis=-1, keepdims=True)
    yc = y - mean
    var = jnp.mean(yc * yc, axis=-1, keepdims=True)
    out[...] = yc * lax.rsqrt(var + 1e-5) * lns[...] + lnb[...] + nf[...]


def _tc_mlp(part, nfeat, W1, b1, W2, b2, ln_scale, ln_bias):
    blk = 2000
    grid = (N_NODES // blk,)
    full = lambda shape: pl.BlockSpec(shape, lambda i: (0, 0))
    return pl.pallas_call(
        _mlp_body,
        grid=grid,
        in_specs=[
            pl.BlockSpec((blk, D), lambda i: (i, 0)),
            pl.BlockSpec((blk, D), lambda i: (i + N_NODES // blk, 0)),
            pl.BlockSpec((blk, D), lambda i: (i, 0)),
            full((2 * D, D)),
            full((1, D)),
            full((D, D)),
            full((1, D)),
            full((1, D)),
            full((1, D)),
        ],
        out_specs=pl.BlockSpec((blk, D), lambda i: (i, 0)),
        out_shape=jax.ShapeDtypeStruct((N_NODES, D), jnp.float32),
    )(part, part, nfeat, W1, b1, W2, b2, ln_scale, ln_bias)


def kernel(efeat, nfeat, edge_index, W1, b1, W2, b2, ln_scale, ln_bias):
    dst3d = edge_index.reshape(2, NCHUNK, CHUNK)
    part = _sc_segment_sum()(efeat, dst3d)
    nfeat_new = _tc_mlp(part, nfeat, W1, b1.reshape(1, D), W2,
                        b2.reshape(1, D), ln_scale.reshape(1, D),
                        ln_bias.reshape(1, D))
    return (_tc_copy(efeat), nfeat_new)


# speedup vs baseline: 1.3629x; 1.2434x over previous
import functools

import jax
import jax.numpy as jnp
from jax import lax
from jax.experimental import pallas as pl
from jax.experimental.pallas import tpu as pltpu
from jax.experimental.pallas import tpu_sc as plsc

N_NODES = 10000
N_EDGES = 320000
D = 128

CHUNK = 128
NCHUNK = N_EDGES // CHUNK
NC = 2
NS = 16
NW = NC * NS
ROWS_PER_W = 80
NROWS_PAD = NW * ROWS_PER_W
NSTEP = ROWS_PER_W
DEPTH = 3
ROWS_PER_TILE = 624
TAIL_ROW0 = NS * ROWS_PER_TILE
TAIL_ROWS = N_NODES - TAIL_ROW0


def _sc_segment_sum_body(efeat_hbm, dst3d_hbm, out_hbm, eout_hbm,
                         acc, bufs, idxs, sems, isems, ssems, osems):
    c = lax.axis_index("c")
    s = lax.axis_index("s")
    wid = c * NS + s

    lo_row = wid * ROWS_PER_W

    def gather(g):
        k = g % DEPTH
        row = lo_row + g
        pltpu.async_copy(dst3d_hbm.at[1, pl.ds(row, 1)], idxs[k], isems[k])
        pltpu.async_copy(efeat_hbm.at[pl.ds(row * CHUNK, CHUNK)],
                         bufs[k], sems[k])

    def wait(g):
        k = g % DEPTH
        row = lo_row + g
        pltpu.make_async_copy(dst3d_hbm.at[1, pl.ds(row, 1)], idxs[k],
                              isems[k]).wait()
        pltpu.make_async_copy(efeat_hbm.at[pl.ds(row * CHUNK, CHUNK)],
                              bufs[k], sems[k]).wait()

    gather(0)

    zbuf = bufs[DEPTH - 1]

    def zfill(r, carry):
        for j in range(D // 16):
            zbuf[r, pl.ds(16 * j, 16)] = jnp.zeros((16,), jnp.float32)
        return carry

    lax.fori_loop(0, CHUNK, zfill, 0)
    row0 = s * ROWS_PER_TILE
    for b in range(ROWS_PER_TILE // CHUNK):
        pltpu.sync_copy(zbuf, acc.at[pl.ds(row0 + b * CHUNK, CHUNK)])
    rem = ROWS_PER_TILE % CHUNK
    pltpu.sync_copy(zbuf.at[pl.ds(0, rem)],
                    acc.at[pl.ds(row0 + ROWS_PER_TILE - rem, rem)])

    @pl.when(s == 0)
    def _():
        pltpu.sync_copy(zbuf.at[pl.ds(0, TAIL_ROWS)],
                        acc.at[pl.ds(TAIL_ROW0, TAIL_ROWS)])

    plsc.subcore_barrier()

    def scatter_start(g):
        k = g % DEPTH
        row = lo_row + g
        pltpu.async_copy(bufs[k], acc.at[idxs[k].at[0]], ssems[k], add=True)
        pltpu.async_copy(bufs[k], eout_hbm.at[pl.ds(row * CHUNK, CHUNK)],
                         osems[k])

    def scatter_wait(g):
        k = g % DEPTH
        row = lo_row + g
        pltpu.make_async_copy(bufs[k], acc.at[idxs[k].at[0]],
                              ssems[k]).wait()
        pltpu.make_async_copy(bufs[k], eout_hbm.at[pl.ds(row * CHUNK, CHUNK)],
                              osems[k]).wait()

    def active(g):
        return lo_row + g < NCHUNK

    for g in range(NSTEP):
        if g >= 2:
            @pl.when(active(g - 2))
            def _(g=g):
                scatter_wait(g - 2)

        if g + 1 < NSTEP:
            @pl.when(active(g + 1))
            def _(g=g):
                gather(g + 1)

        @pl.when(active(g))
        def _(g=g):
            wait(g)
            scatter_start(g)

    for t in range(max(0, NSTEP - 2), NSTEP):
        @pl.when(active(t))
        def _(t=t):
            scatter_wait(t)

    plsc.subcore_barrier()

    pltpu.sync_copy(acc.at[pl.ds(row0, ROWS_PER_TILE)],
                    out_hbm.at[pl.ds(c * N_NODES + row0, ROWS_PER_TILE)])

    @pl.when(s == 0)
    def _():
        pltpu.sync_copy(acc.at[pl.ds(TAIL_ROW0, TAIL_ROWS)],
                        out_hbm.at[pl.ds(c * N_NODES + TAIL_ROW0, TAIL_ROWS)])


@functools.cache
def _sc_segment_sum():
    return pl.kernel(
        _sc_segment_sum_body,
        mesh=plsc.VectorSubcoreMesh(core_axis_name="c", subcore_axis_name="s",
                                    num_cores=NC, num_subcores=NS),
        out_type=(jax.ShapeDtypeStruct((NC * N_NODES, D), jnp.float32),
                  jax.ShapeDtypeStruct((N_EDGES, D), jnp.float32)),
        scratch_types=[
            pltpu.VMEM_SHARED((N_NODES, D), jnp.float32),
            tuple(pltpu.VMEM((CHUNK, D), jnp.float32) for _ in range(DEPTH)),
            tuple(pltpu.VMEM((1, CHUNK), jnp.int32) for _ in range(DEPTH)),
            tuple(pltpu.SemaphoreType.DMA for _ in range(DEPTH)),
            tuple(pltpu.SemaphoreType.DMA for _ in range(DEPTH)),
            tuple(pltpu.SemaphoreType.DMA for _ in range(DEPTH)),
            tuple(pltpu.SemaphoreType.DMA for _ in range(DEPTH)),
        ],
    )


def _mlp_body(p0, p1, nf, w1, b1, w2, b2, lns, lnb, out):
    agg = p0[...] + p1[...]
    x = (jnp.dot(agg, w1[0:D, :], preferred_element_type=jnp.float32)
         + jnp.dot(nf[...], w1[D:2 * D, :], preferred_element_type=jnp.float32)
         + b1[...])
    h = x * jax.nn.sigmoid(x)
    y = jnp.dot(h, w2[...], preferred_element_type=jnp.float32) + b2[...]
    mean = jnp.mean(y, axis=-1, keepdims=True)
    yc = y - mean
    var = jnp.mean(yc * yc, axis=-1, keepdims=True)
    out[...] = yc * lax.rsqrt(var + 1e-5) * lns[...] + lnb[...] + nf[...]


def _tc_mlp(part, nfeat, W1, b1, W2, b2, ln_scale, ln_bias):
    blk = 2000
    grid = (N_NODES // blk,)
    full = lambda shape: pl.BlockSpec(shape, lambda i: (0, 0))
    return pl.pallas_call(
        _mlp_body,
        grid=grid,
        in_specs=[
            pl.BlockSpec((blk, D), lambda i: (i, 0)),
            pl.BlockSpec((blk, D), lambda i: (i + N_NODES // blk, 0)),
            pl.BlockSpec((blk, D), lambda i: (i, 0)),
            full((2 * D, D)),
            full((1, D)),
            full((D, D)),
            full((1, D)),
            full((1, D)),
            full((1, D)),
        ],
        out_specs=pl.BlockSpec((blk, D), lambda i: (i, 0)),
        out_shape=jax.ShapeDtypeStruct((N_NODES, D), jnp.float32),
    )(part, part, nfeat, W1, b1, W2, b2, ln_scale, ln_bias)


def kernel(efeat, nfeat, edge_index, W1, b1, W2, b2, ln_scale, ln_bias):
    dst3d = edge_index.reshape(2, NCHUNK, CHUNK)
    part, efeat_out = _sc_segment_sum()(efeat, dst3d)
    nfeat_new = _tc_mlp(part, nfeat, W1, b1.reshape(1, D), W2,
                        b2.reshape(1, D), ln_scale.reshape(1, D),
                        ln_bias.reshape(1, D))
    return (efeat_out, nfeat_new)


# speedup vs baseline: 1.3669x; 1.0029x over previous
import functools

import jax
import jax.numpy as jnp
from jax import lax
from jax.experimental import pallas as pl
from jax.experimental.pallas import tpu as pltpu
from jax.experimental.pallas import tpu_sc as plsc

N_NODES = 10000
N_EDGES = 320000
D = 128

CHUNK = 128
NCHUNK = N_EDGES // CHUNK
NC = 2
NS = 16
NW = NC * NS
ROWS_PER_W = 80
NROWS_PAD = NW * ROWS_PER_W
NSTEP = ROWS_PER_W
DEPTH = 3
ROWS_PER_TILE = 624
TAIL_ROW0 = NS * ROWS_PER_TILE
TAIL_ROWS = N_NODES - TAIL_ROW0


def _sc_segment_sum_body(efeat_hbm, ei_hbm, out_hbm, eout_hbm,
                         acc, bufs, idxs, sems, isems, ssems, osems):
    c = lax.axis_index("c")
    s = lax.axis_index("s")
    wid = c * NS + s

    lo_row = wid * ROWS_PER_W

    def gather(g):
        k = g % DEPTH
        row = lo_row + g
        pltpu.async_copy(ei_hbm.at[1, pl.ds(row * CHUNK, CHUNK)],
                         idxs[k], isems[k])
        pltpu.async_copy(efeat_hbm.at[pl.ds(row * CHUNK, CHUNK)],
                         bufs[k], sems[k])

    def wait(g):
        k = g % DEPTH
        row = lo_row + g
        pltpu.make_async_copy(ei_hbm.at[1, pl.ds(row * CHUNK, CHUNK)],
                              idxs[k], isems[k]).wait()
        pltpu.make_async_copy(efeat_hbm.at[pl.ds(row * CHUNK, CHUNK)],
                              bufs[k], sems[k]).wait()

    gather(0)

    zbuf = bufs[DEPTH - 1]

    def zfill(r, carry):
        for j in range(D // 16):
            zbuf[r, pl.ds(16 * j, 16)] = jnp.zeros((16,), jnp.float32)
        return carry

    lax.fori_loop(0, CHUNK, zfill, 0)
    row0 = s * ROWS_PER_TILE
    for b in range(ROWS_PER_TILE // CHUNK):
        pltpu.sync_copy(zbuf, acc.at[pl.ds(row0 + b * CHUNK, CHUNK)])
    rem = ROWS_PER_TILE % CHUNK
    pltpu.sync_copy(zbuf.at[pl.ds(0, rem)],
                    acc.at[pl.ds(row0 + ROWS_PER_TILE - rem, rem)])

    @pl.when(s == 0)
    def _():
        pltpu.sync_copy(zbuf.at[pl.ds(0, TAIL_ROWS)],
                        acc.at[pl.ds(TAIL_ROW0, TAIL_ROWS)])

    plsc.subcore_barrier()

    def scatter_start(g):
        k = g % DEPTH
        row = lo_row + g
        pltpu.async_copy(bufs[k], acc.at[idxs[k]], ssems[k], add=True)
        pltpu.async_copy(bufs[k], eout_hbm.at[pl.ds(row * CHUNK, CHUNK)],
                         osems[k])

    def scatter_wait(g):
        k = g % DEPTH
        row = lo_row + g
        pltpu.make_async_copy(bufs[k], acc.at[idxs[k]],
                              ssems[k]).wait()
        pltpu.make_async_copy(bufs[k], eout_hbm.at[pl.ds(row * CHUNK, CHUNK)],
                              osems[k]).wait()

    def active(g):
        return lo_row + g < NCHUNK

    for g in range(NSTEP):
        if g >= 2:
            @pl.when(active(g - 2))
            def _(g=g):
                scatter_wait(g - 2)

        if g + 1 < NSTEP:
            @pl.when(active(g + 1))
            def _(g=g):
                gather(g + 1)

        @pl.when(active(g))
        def _(g=g):
            wait(g)
            scatter_start(g)

    for t in range(max(0, NSTEP - 2), NSTEP):
        @pl.when(active(t))
        def _(t=t):
            scatter_wait(t)

    plsc.subcore_barrier()

    pltpu.sync_copy(acc.at[pl.ds(row0, ROWS_PER_TILE)],
                    out_hbm.at[pl.ds(c * N_NODES + row0, ROWS_PER_TILE)])

    @pl.when(s == 0)
    def _():
        pltpu.sync_copy(acc.at[pl.ds(TAIL_ROW0, TAIL_ROWS)],
                        out_hbm.at[pl.ds(c * N_NODES + TAIL_ROW0, TAIL_ROWS)])


@functools.cache
def _sc_segment_sum():
    return pl.kernel(
        _sc_segment_sum_body,
        mesh=plsc.VectorSubcoreMesh(core_axis_name="c", subcore_axis_name="s",
                                    num_cores=NC, num_subcores=NS),
        out_type=(jax.ShapeDtypeStruct((NC * N_NODES, D), jnp.float32),
                  jax.ShapeDtypeStruct((N_EDGES, D), jnp.float32)),
        scratch_types=[
            pltpu.VMEM_SHARED((N_NODES, D), jnp.float32),
            tuple(pltpu.VMEM((CHUNK, D), jnp.float32) for _ in range(DEPTH)),
            tuple(pltpu.VMEM((CHUNK,), jnp.int32) for _ in range(DEPTH)),
            tuple(pltpu.SemaphoreType.DMA for _ in range(DEPTH)),
            tuple(pltpu.SemaphoreType.DMA for _ in range(DEPTH)),
            tuple(pltpu.SemaphoreType.DMA for _ in range(DEPTH)),
            tuple(pltpu.SemaphoreType.DMA for _ in range(DEPTH)),
        ],
    )


def _mlp_body(p0, p1, nf, w1, b1, w2, b2, lns, lnb, out):
    agg = p0[...] + p1[...]
    x = (jnp.dot(agg, w1[0:D, :], preferred_element_type=jnp.float32)
         + jnp.dot(nf[...], w1[D:2 * D, :], preferred_element_type=jnp.float32)
         + b1[...])
    h = x * jax.nn.sigmoid(x)
    y = jnp.dot(h, w2[...], preferred_element_type=jnp.float32) + b2[...]
    mean = jnp.mean(y, axis=-1, keepdims=True)
    yc = y - mean
    var = jnp.mean(yc * yc, axis=-1, keepdims=True)
    out[...] = yc * lax.rsqrt(var + 1e-5) * lns[...] + lnb[...] + nf[...]


def _tc_mlp(part, nfeat, W1, b1, W2, b2, ln_scale, ln_bias):
    blk = 2000
    grid = (N_NODES // blk,)
    full = lambda shape: pl.BlockSpec(shape, lambda i: (0, 0))
    return pl.pallas_call(
        _mlp_body,
        grid=grid,
        in_specs=[
            pl.BlockSpec((blk, D), lambda i: (i, 0)),
            pl.BlockSpec((blk, D), lambda i: (i + N_NODES // blk, 0)),
            pl.BlockSpec((blk, D), lambda i: (i, 0)),
            full((2 * D, D)),
            full((1, D)),
            full((D, D)),
            full((1, D)),
            full((1, D)),
            full((1, D)),
        ],
        out_specs=pl.BlockSpec((blk, D), lambda i: (i, 0)),
        out_shape=jax.ShapeDtypeStruct((N_NODES, D), jnp.float32),
    )(part, part, nfeat, W1, b1, W2, b2, ln_scale, ln_bias)


def kernel(efeat, nfeat, edge_index, W1, b1, W2, b2, ln_scale, ln_bias):
    part, efeat_out = _sc_segment_sum()(efeat, edge_index)
    nfeat_new = _tc_mlp(part, nfeat, W1, b1.reshape(1, D), W2,
                        b2.reshape(1, D), ln_scale.reshape(1, D),
                        ln_bias.reshape(1, D))
    return (efeat_out, nfeat_new)
